# Initial kernel scaffold; baseline (speedup 1.0000x reference)
import jax
import jax.numpy as jnp
from jax.experimental import pallas as pl


def kernel(X, user_table, item_table, W1, b1, W2, b2):
    raise NotImplementedError("write your pallas kernel here")


# speedup vs baseline: 1.1464x; 1.1464x over previous
import functools

import jax
import jax.numpy as jnp
from jax import lax
from jax.experimental import pallas as pl
from jax.experimental.pallas import tpu as pltpu
from jax.experimental.pallas import tpu_sc as plsc

BATCH = 16384
EMB = 64
N_NUM = 8
N_ACT = 256
RATING_MIN = 1.0
RATING_MAX = 5.0

_info = plsc.get_sparse_core_info()
_NC, _NS = _info.num_cores, _info.num_subcores
_NW = _NC * _NS
_BPW = BATCH // _NW


def _sc_gather_body(ut_hbm, it_hbm, uidx_hbm, iidx_hbm, eu_hbm, ei_hbm,
                    uidx_v, urows_v, iidx_v, irows_v, sem_u, sem_i):
    wid = lax.axis_index("s") * _NC + lax.axis_index("c")
    base = wid * _BPW
    pltpu.sync_copy(uidx_hbm.at[pl.ds(base, _BPW)], uidx_v)
    pltpu.sync_copy(iidx_hbm.at[pl.ds(base, _BPW)], iidx_v)
    cu = pltpu.async_copy(ut_hbm.at[uidx_v], urows_v, sem_u)
    ci = pltpu.async_copy(it_hbm.at[iidx_v], irows_v, sem_i)
    cu.wait()
    ci.wait()
    pltpu.sync_copy(urows_v, eu_hbm.at[pl.ds(base, _BPW)])
    pltpu.sync_copy(irows_v, ei_hbm.at[pl.ds(base, _BPW)])


_sc_gather = functools.partial(
    pl.kernel,
    mesh=plsc.VectorSubcoreMesh(core_axis_name="c", subcore_axis_name="s"),
    compiler_params=pltpu.CompilerParams(use_tc_tiling_on_sc=False),
    out_type=[
        jax.ShapeDtypeStruct((BATCH, EMB), jnp.float32),
        jax.ShapeDtypeStruct((BATCH, EMB), jnp.float32),
    ],
    scratch_types=[
        pltpu.VMEM((_BPW,), jnp.int32),
        pltpu.VMEM((_BPW, EMB), jnp.float32),
        pltpu.VMEM((_BPW,), jnp.int32),
        pltpu.VMEM((_BPW, EMB), jnp.float32),
        pltpu.SemaphoreType.DMA,
        pltpu.SemaphoreType.DMA,
    ],
)(_sc_gather_body)


_BT = 2048


def _mlp_body(eu_ref, ei_ref, nf_ref, w1u_ref, w1i_ref, w1n_ref, b1_ref,
              w2_ref, b2_ref, out_ref):
    h = jnp.dot(eu_ref[...], w1u_ref[...], preferred_element_type=jnp.float32)
    h += jnp.dot(ei_ref[...], w1i_ref[...], preferred_element_type=jnp.float32)
    h += jnp.dot(nf_ref[...], w1n_ref[...], preferred_element_type=jnp.float32)
    h += b1_ref[...]
    h = jnp.maximum(h, 0.0)
    o = jnp.dot(h, w2_ref[...], preferred_element_type=jnp.float32)
    o += b2_ref[...]
    o = 1.0 / (1.0 + jnp.exp(-o))
    out_ref[...] = o * (RATING_MAX - RATING_MIN) + RATING_MIN


def _mlp(eu, ei, nf, w1u, w1i, w1n, b1, w2, b2):
    grid = (BATCH // _BT,)
    bspec_b = lambda shape: pl.BlockSpec((_BT,) + shape[1:],
                                         lambda i: (i,) + (0,) * (len(shape) - 1))
    full = lambda shape: pl.BlockSpec(shape, lambda i: (0,) * len(shape))
    return pl.pallas_call(
        _mlp_body,
        grid=grid,
        in_specs=[
            bspec_b(eu.shape), bspec_b(ei.shape), bspec_b(nf.shape),
            full(w1u.shape), full(w1i.shape), full(w1n.shape), full(b1.shape),
            full(w2.shape), full(b2.shape),
        ],
        out_specs=pl.BlockSpec((_BT, 1), lambda i: (i, 0)),
        out_shape=jax.ShapeDtypeStruct((BATCH, 1), jnp.float32),
    )(eu, ei, nf, w1u, w1i, w1n, b1, w2, b2)


def kernel(X, user_table, item_table, W1, b1, W2, b2):
    uidx = X[:, 0]
    iidx = X[:, 1]
    nf = X[:, 2:].astype(jnp.float32)
    eu, ei = _sc_gather(user_table, item_table, uidx, iidx)
    w1u = W1[:EMB]
    w1i = W1[EMB:2 * EMB]
    w1n = W1[2 * EMB:]
    return _mlp(eu, ei, nf, w1u, w1i, w1n, b1.reshape(1, N_ACT), W2,
                b2.reshape(1, 1))
